# baseline (speedup 1.0000x reference)
import jax
import jax.numpy as jnp
from jax.experimental import pallas as pl

_B = 2048
_C = 512
_R = 64
_K = _C * _R
_NOUT = 8
_BT = 64


def _mm_kernel(x_ref, w_ref, o_ref):
    o_ref[...] = jax.lax.dot_general(
        x_ref[...], w_ref[...],
        (((1,), (0,)), ((), ())),
        preferred_element_type=jnp.float32,
    )


def kernel(x, start_boundaries, W_loc, b_loc, W_conf, b_conf):
    del start_boundaries
    xf = x.reshape(_B, _K)
    Wc = jnp.concatenate([W_loc, W_conf], axis=0)
    Wc = jnp.pad(Wc, ((0, _NOUT - 5), (0, 0)))
    w_rep = jnp.repeat(Wc.T / _R, _R, axis=0)

    out = pl.pallas_call(
        _mm_kernel,
        grid=(_B // _BT,),
        in_specs=[
            pl.BlockSpec((_BT, _K), lambda i: (i, 0)),
            pl.BlockSpec((_K, _NOUT), lambda i: (0, 0)),
        ],
        out_specs=pl.BlockSpec((_BT, _NOUT), lambda i: (i, 0)),
        out_shape=jax.ShapeDtypeStruct((_B, _NOUT), jnp.float32),
    )(xf, w_rep)

    loc = out[:, :2] + b_loc
    conf = out[:, 2:5] + b_conf
    return (loc, conf)


# speedup vs baseline: 1.1060x; 1.1060x over previous
import jax
import jax.numpy as jnp
from jax.experimental import pallas as pl

_B = 2048
_C = 512
_R = 64
_K = _C * _R
_NOUT = 8
_BT = 64


def _red_kernel(x_ref, w_ref, o_ref):
    s = jnp.sum(x_ref[...], axis=2)
    o_ref[...] = jax.lax.dot_general(
        s, w_ref[...],
        (((1,), (0,)), ((), ())),
        preferred_element_type=jnp.float32,
    )


def kernel(x, start_boundaries, W_loc, b_loc, W_conf, b_conf):
    del start_boundaries
    x3 = x.reshape(_B, _C, _R)
    Wc = jnp.concatenate([W_loc, W_conf], axis=0)
    Wc = jnp.pad(Wc, ((0, _NOUT - 5), (0, 0)))
    w = Wc.T / _R

    out = pl.pallas_call(
        _red_kernel,
        grid=(_B // _BT,),
        in_specs=[
            pl.BlockSpec((_BT, _C, _R), lambda i: (i, 0, 0)),
            pl.BlockSpec((_C, _NOUT), lambda i: (0, 0)),
        ],
        out_specs=pl.BlockSpec((_BT, _NOUT), lambda i: (i, 0)),
        out_shape=jax.ShapeDtypeStruct((_B, _NOUT), jnp.float32),
    )(x3, w)

    loc = out[:, :2] + b_loc
    conf = out[:, 2:5] + b_conf
    return (loc, conf)
